# baseline (device time: 6633 ns/iter reference)
import jax
import jax.numpy as jnp
from jax import lax
from jax.experimental import pallas as pl
from jax.experimental.pallas import tpu as pltpu


def kernel(x, dy, gamma):
    m, d = x.shape

    def body(
        x_hbm,
        dy_hbm,
        out_hbm,
        xv_ref,
        dyv_ref,
        local_ref,
        comm_ref,
        sum_ref,
        copy_sems,
        out_sem,
        send_sem,
        recv_sem,
    ):
        my_x = lax.axis_index("x")
        my_y = lax.axis_index("y")
        my_z = lax.axis_index("z")
        peer = (my_x, 1 - my_y, my_z)

        barrier_sem = pltpu.get_barrier_semaphore()
        pl.semaphore_signal(
            barrier_sem,
            inc=1,
            device_id=(my_x, my_y, my_z),
            device_id_type=pl.DeviceIdType.MESH,
        )
        pl.semaphore_wait(barrier_sem, 1)

        cp_x = pltpu.make_async_copy(x_hbm, xv_ref, copy_sems.at[0])
        cp_dy = pltpu.make_async_copy(dy_hbm, dyv_ref, copy_sems.at[1])
        cp_x.start()
        cp_dy.start()
        cp_x.wait()
        cp_dy.wait()

        xv = xv_ref[:, :]
        dyv = dyv_ref[:, :]
        mu = jnp.mean(xv, axis=1, keepdims=True)
        xc = xv - mu
        var = jnp.mean(xc * xc, axis=1, keepdims=True)
        rstd = lax.rsqrt(var + 1e-5)
        xhat = xc * rstd
        local_ref[0:1, :] = jnp.sum(dyv * xhat, axis=0, keepdims=True)
        local_ref[1:2, :] = jnp.sum(dyv, axis=0, keepdims=True)

        rdma = pltpu.make_async_remote_copy(
            src_ref=local_ref,
            dst_ref=comm_ref,
            send_sem=send_sem,
            recv_sem=recv_sem,
            device_id=peer,
            device_id_type=pl.DeviceIdType.MESH,
        )
        rdma.start()
        rdma.wait_recv()

        sum_ref[:, :] = local_ref[:, :] + comm_ref[:, :]
        cp_out = pltpu.make_async_copy(sum_ref, out_hbm, out_sem)
        cp_out.start()
        rdma.wait_send()
        cp_out.wait()

    out = pl.pallas_call(
        body,
        out_shape=jax.ShapeDtypeStruct((2, d), jnp.float32),
        in_specs=[
            pl.BlockSpec(memory_space=pl.ANY),
            pl.BlockSpec(memory_space=pl.ANY),
        ],
        out_specs=pl.BlockSpec(memory_space=pltpu.MemorySpace.HBM),
        scratch_shapes=[
            pltpu.VMEM((m, d), jnp.float32),
            pltpu.VMEM((m, d), jnp.float32),
            pltpu.VMEM((2, d), jnp.float32),
            pltpu.VMEM((2, d), jnp.float32),
            pltpu.VMEM((2, d), jnp.float32),
            pltpu.SemaphoreType.DMA((2,)),
            pltpu.SemaphoreType.DMA,
            pltpu.SemaphoreType.DMA,
            pltpu.SemaphoreType.DMA,
        ],
        compiler_params=pltpu.CompilerParams(collective_id=0),
    )(
        pltpu.with_memory_space_constraint(x, pltpu.MemorySpace.HBM),
        pltpu.with_memory_space_constraint(dy, pltpu.MemorySpace.HBM),
    )
    return out


# device time: 6482 ns/iter; 1.0233x vs baseline; 1.0233x over previous
import jax
import jax.numpy as jnp
from jax import lax
from jax.experimental import pallas as pl
from jax.experimental.pallas import tpu as pltpu

N_BLK = 4


def kernel(x, dy, gamma):
    m, d = x.shape
    mb = m // N_BLK

    def body(
        x_hbm,
        dy_hbm,
        out_hbm,
        xv_ref,
        dyv_ref,
        local_ref,
        comm_ref,
        sum_ref,
        copy_sems,
        out_sem,
        send_sem,
        recv_sem,
    ):
        my_x = lax.axis_index("x")
        my_y = lax.axis_index("y")
        my_z = lax.axis_index("z")
        peer = (my_x, 1 - my_y, my_z)

        barrier_sem = pltpu.get_barrier_semaphore()
        pl.semaphore_signal(
            barrier_sem,
            inc=1,
            device_id=peer,
            device_id_type=pl.DeviceIdType.MESH,
        )

        cps = []
        for k in range(N_BLK):
            rows = pl.ds(k * mb, mb)
            cp_x = pltpu.make_async_copy(
                x_hbm.at[rows], xv_ref.at[rows], copy_sems.at[0, k]
            )
            cp_dy = pltpu.make_async_copy(
                dy_hbm.at[rows], dyv_ref.at[rows], copy_sems.at[1, k]
            )
            cp_x.start()
            cp_dy.start()
            cps.append((cp_x, cp_dy))

        dgamma = jnp.zeros((1, d), jnp.float32)
        dbeta = jnp.zeros((1, d), jnp.float32)
        for k in range(N_BLK):
            cp_x, cp_dy = cps[k]
            cp_x.wait()
            cp_dy.wait()
            rows = pl.ds(k * mb, mb)
            xk = xv_ref[rows, :]
            dyk = dyv_ref[rows, :]
            s1 = jnp.sum(xk, axis=1, keepdims=True)
            s2 = jnp.sum(xk * xk, axis=1, keepdims=True)
            mu = s1 * (1.0 / d)
            var = s2 * (1.0 / d) - mu * mu
            rstd = lax.rsqrt(var + 1e-5)
            b = mu * rstd
            dgamma = dgamma + jnp.sum(
                dyk * (xk * rstd - b), axis=0, keepdims=True
            )
            dbeta = dbeta + jnp.sum(dyk, axis=0, keepdims=True)
        local_ref[0:1, :] = dgamma
        local_ref[1:2, :] = dbeta

        pl.semaphore_wait(barrier_sem, 1)

        rdma = pltpu.make_async_remote_copy(
            src_ref=local_ref,
            dst_ref=comm_ref,
            send_sem=send_sem,
            recv_sem=recv_sem,
            device_id=peer,
            device_id_type=pl.DeviceIdType.MESH,
        )
        rdma.start()
        rdma.wait_recv()

        sum_ref[:, :] = local_ref[:, :] + comm_ref[:, :]
        cp_out = pltpu.make_async_copy(sum_ref, out_hbm, out_sem)
        cp_out.start()
        rdma.wait_send()
        cp_out.wait()

    out = pl.pallas_call(
        body,
        out_shape=jax.ShapeDtypeStruct((2, d), jnp.float32),
        in_specs=[
            pl.BlockSpec(memory_space=pl.ANY),
            pl.BlockSpec(memory_space=pl.ANY),
        ],
        out_specs=pl.BlockSpec(memory_space=pltpu.MemorySpace.HBM),
        scratch_shapes=[
            pltpu.VMEM((m, d), jnp.float32),
            pltpu.VMEM((m, d), jnp.float32),
            pltpu.VMEM((2, d), jnp.float32),
            pltpu.VMEM((2, d), jnp.float32),
            pltpu.VMEM((2, d), jnp.float32),
            pltpu.SemaphoreType.DMA((2, N_BLK)),
            pltpu.SemaphoreType.DMA,
            pltpu.SemaphoreType.DMA,
            pltpu.SemaphoreType.DMA,
        ],
        compiler_params=pltpu.CompilerParams(collective_id=0),
    )(
        pltpu.with_memory_space_constraint(x, pltpu.MemorySpace.HBM),
        pltpu.with_memory_space_constraint(dy, pltpu.MemorySpace.HBM),
    )
    return out


# device time: 6396 ns/iter; 1.0371x vs baseline; 1.0134x over previous
import jax
import jax.numpy as jnp
from jax import lax
from jax.experimental import pallas as pl
from jax.experimental.pallas import tpu as pltpu

N_BLK = 2


def kernel(x, dy, gamma):
    m, d = x.shape
    mb = m // N_BLK

    def body(
        x_hbm,
        dy_hbm,
        out_hbm,
        xv_ref,
        dyv_ref,
        local_ref,
        comm_ref,
        sum_ref,
        copy_sems,
        out_sem,
        send_sem,
        recv_sem,
    ):
        my_x = lax.axis_index("x")
        my_y = lax.axis_index("y")
        my_z = lax.axis_index("z")
        peer = (my_x, 1 - my_y, my_z)

        barrier_sem = pltpu.get_barrier_semaphore()
        pl.semaphore_signal(
            barrier_sem,
            inc=1,
            device_id=peer,
            device_id_type=pl.DeviceIdType.MESH,
        )

        cps = []
        for k in range(N_BLK):
            rows = pl.ds(k * mb, mb)
            cp_x = pltpu.make_async_copy(
                x_hbm.at[rows], xv_ref.at[rows], copy_sems.at[0, k]
            )
            cp_dy = pltpu.make_async_copy(
                dy_hbm.at[rows], dyv_ref.at[rows], copy_sems.at[1, k]
            )
            cp_x.start()
            cp_dy.start()
            cps.append((cp_x, cp_dy))

        dgamma = jnp.zeros((1, d), jnp.float32)
        dbeta = jnp.zeros((1, d), jnp.float32)
        for k in range(N_BLK):
            cp_x, cp_dy = cps[k]
            cp_x.wait()
            cp_dy.wait()
            rows = pl.ds(k * mb, mb)
            xk = xv_ref[rows, :]
            dyk = dyv_ref[rows, :]
            s1 = jnp.sum(xk, axis=1, keepdims=True)
            s2 = jnp.sum(xk * xk, axis=1, keepdims=True)
            mu = s1 * (1.0 / d)
            var = s2 * (1.0 / d) - mu * mu
            rstd = lax.rsqrt(var + 1e-5)
            b = mu * rstd
            dgamma = dgamma + jnp.sum(
                dyk * (xk * rstd - b), axis=0, keepdims=True
            )
            dbeta = dbeta + jnp.sum(dyk, axis=0, keepdims=True)
        local_ref[0:1, :] = dgamma
        local_ref[1:2, :] = dbeta

        pl.semaphore_wait(barrier_sem, 1)

        rdma = pltpu.make_async_remote_copy(
            src_ref=local_ref,
            dst_ref=comm_ref,
            send_sem=send_sem,
            recv_sem=recv_sem,
            device_id=peer,
            device_id_type=pl.DeviceIdType.MESH,
        )
        rdma.start()
        rdma.wait_recv()

        sum_ref[:, :] = local_ref[:, :] + comm_ref[:, :]
        cp_out = pltpu.make_async_copy(sum_ref, out_hbm, out_sem)
        cp_out.start()
        rdma.wait_send()
        cp_out.wait()

    out = pl.pallas_call(
        body,
        out_shape=jax.ShapeDtypeStruct((2, d), jnp.float32),
        in_specs=[
            pl.BlockSpec(memory_space=pl.ANY),
            pl.BlockSpec(memory_space=pl.ANY),
        ],
        out_specs=pl.BlockSpec(memory_space=pltpu.MemorySpace.HBM),
        scratch_shapes=[
            pltpu.VMEM((m, d), jnp.float32),
            pltpu.VMEM((m, d), jnp.float32),
            pltpu.VMEM((2, d), jnp.float32),
            pltpu.VMEM((2, d), jnp.float32),
            pltpu.VMEM((2, d), jnp.float32),
            pltpu.SemaphoreType.DMA((2, N_BLK)),
            pltpu.SemaphoreType.DMA,
            pltpu.SemaphoreType.DMA,
            pltpu.SemaphoreType.DMA,
        ],
        compiler_params=pltpu.CompilerParams(collective_id=0),
    )(
        pltpu.with_memory_space_constraint(x, pltpu.MemorySpace.HBM),
        pltpu.with_memory_space_constraint(dy, pltpu.MemorySpace.HBM),
    )
    return out
